# baseline (speedup 1.0000x reference)
import functools

import jax
import jax.numpy as jnp
from jax import lax
from jax.experimental import pallas as pl
from jax.experimental.pallas import tpu as pltpu
from jax.experimental.pallas import tpu_sc as plsc

_NC = 2
_NS = 16
_L = 16
_IC = 128


@functools.lru_cache(maxsize=None)
def _build(S, D):
    slabs = S // (_NC * _NS)
    span = S + slabs - 1
    chunks = -(-span // _IC)
    rows = chunks * _IC

    mesh = plsc.VectorSubcoreMesh(core_axis_name="c", subcore_axis_name="s")

    @functools.partial(
        pl.kernel,
        out_type=jax.ShapeDtypeStruct((S * S, D), jnp.float32),
        mesh=mesh,
        scratch_types=[
            pltpu.VMEM((chunks, _IC), jnp.int32),
            pltpu.VMEM((rows, D), jnp.float32),
            pltpu.SemaphoreType.DMA,
        ],
    )
    def k(table, out, idx_v, buf_v, sem):
        c = lax.axis_index("c")
        s = lax.axis_index("s")
        row0 = c * (S // _NC) + s * slabs

        top = row0 + (S + slabs - 2)
        for j in range(chunks):
            idx_row = idx_v.at[j]
            for b in range(_IC // _L):
                t0 = j * _IC + b * _L
                v = (top - t0) - lax.iota(jnp.int32, _L)
                idx_row[pl.ds(b * _L, _L)] = jnp.maximum(v, 0)

        gathers = [
            pltpu.async_copy(table.at[idx_v.at[j]],
                             buf_v.at[pl.ds(j * _IC, _IC)], sem)
            for j in range(chunks)
        ]
        for g in gathers:
            g.wait()

        copies = [
            pltpu.async_copy(buf_v.at[pl.ds(slabs - 1 - r, S)],
                             out.at[pl.ds((row0 + r) * S, S)], sem)
            for r in range(slabs)
        ]
        for cp in copies:
            cp.wait()

    return k


def kernel(rel_pos_emb, seq_len):
    del seq_len
    T, D = rel_pos_emb.shape
    S = (T + 1) // 2
    out2d = _build(S, D)(rel_pos_emb)
    return out2d.reshape(S, S, D)
